# baseline (speedup 1.0000x reference)
import functools

import jax
import jax.numpy as jnp
from jax import lax
from jax.experimental import pallas as pl
from jax.experimental.pallas import tpu as pltpu
from jax.experimental.pallas import tpu_sc as plsc

N = 50000
E = 800000
H = 64
G = 16

NC = 2
NS = 16
NW = NC * NS

E_PAD = 819200
PADE = E_PAD - E

NPAD = 50048
ZR = NPAD // NS

L1 = 800
EW1 = E_PAD // NW
CH1 = EW1 // (2 * L1)

L2 = 200
EW2 = E_PAD // NS
CH2 = EW2 // (2 * L2)

R = 2000
NB = N // R

_MESH = plsc.VectorSubcoreMesh(
    core_axis_name="c", subcore_axis_name="s", num_cores=NC, num_subcores=NS)

_SC_PARAMS = pltpu.CompilerParams(use_tc_tiling_on_sc=False)


@functools.partial(
    pl.kernel,
    out_type=jax.ShapeDtypeStruct((NC * NPAD, 8), jnp.float32),
    mesh=_MESH,
    scratch_types=[
        pltpu.VMEM((L1,), jnp.int32),
        pltpu.VMEM((L1,), jnp.int32),
        pltpu.VMEM((L1,), jnp.int32),
        pltpu.VMEM((L1,), jnp.int32),
        pltpu.VMEM((L1, 8), jnp.float32),
        pltpu.VMEM((L1, 8), jnp.float32),
        pltpu.VMEM_SHARED((NPAD, 8), jnp.float32),
        pltpu.SemaphoreType.DMA,
        pltpu.SemaphoreType.DMA,
    ],
    compiler_params=_SC_PARAMS,
)
def _sc_agg8(xa_hbm, srcs_hbm, dsts_hbm, zeros_hbm, out_hbm,
             src0, src1, dst0, dst1, rows0, rows1, acc, sem, sem2):
  c = lax.axis_index("c")
  s = lax.axis_index("s")
  wid = c * NS + s
  pltpu.sync_copy(zeros_hbm.at[pl.ds(s * ZR, ZR)], acc.at[pl.ds(s * ZR, ZR)])
  plsc.subcore_barrier()
  base = wid * EW1

  def chunk(i, carry):
    e0 = base + (2 * i) * L1
    pltpu.sync_copy(srcs_hbm.at[pl.ds(e0, L1)], src0)
    pltpu.sync_copy(dsts_hbm.at[pl.ds(e0, L1)], dst0)
    g0 = pltpu.async_copy(xa_hbm.at[src0], rows0, sem)
    pltpu.sync_copy(srcs_hbm.at[pl.ds(e0 + L1, L1)], src1)
    pltpu.sync_copy(dsts_hbm.at[pl.ds(e0 + L1, L1)], dst1)
    g1 = pltpu.async_copy(xa_hbm.at[src1], rows1, sem)
    g0.wait()
    s0 = pltpu.async_copy(rows0, acc.at[dst0], sem2, add=True)
    g1.wait()
    s1 = pltpu.async_copy(rows1, acc.at[dst1], sem2, add=True)
    s0.wait()
    s1.wait()
    return carry

  lax.fori_loop(0, CH1, chunk, 0)
  plsc.subcore_barrier()
  pltpu.sync_copy(acc.at[pl.ds(s * ZR, ZR)],
                  out_hbm.at[pl.ds(c * NPAD + s * ZR, ZR)])


@functools.partial(
    pl.kernel,
    out_type=jax.ShapeDtypeStruct((NC * NPAD, 32), jnp.float32),
    mesh=_MESH,
    scratch_types=[
        pltpu.VMEM((L2,), jnp.int32),
        pltpu.VMEM((L2,), jnp.int32),
        pltpu.VMEM((L2,), jnp.int32),
        pltpu.VMEM((L2,), jnp.int32),
        pltpu.VMEM((L2, 64), jnp.float32),
        pltpu.VMEM((L2, 64), jnp.float32),
        pltpu.VMEM_SHARED((NPAD, 32), jnp.float32),
        pltpu.SemaphoreType.DMA,
        pltpu.SemaphoreType.DMA,
    ],
    compiler_params=_SC_PARAMS,
)
def _sc_agg32(h_hbm, srcs_hbm, dsts_hbm, zeros_hbm, out_hbm,
              src0, src1, dst0, dst1, rows0, rows1, acc, sem, sem2):
  c = lax.axis_index("c")
  s = lax.axis_index("s")
  off = c * N
  pltpu.sync_copy(zeros_hbm.at[pl.ds(s * ZR, ZR)], acc.at[pl.ds(s * ZR, ZR)])
  plsc.subcore_barrier()
  base = s * EW2

  def add_off(ref):
    for k in range(L2 // 16):
      ref[pl.ds(k * 16, 16)] = ref[pl.ds(k * 16, 16)] + off

  def chunk(i, carry):
    e0 = base + (2 * i) * L2
    pltpu.sync_copy(srcs_hbm.at[pl.ds(e0, L2)], src0)
    pltpu.sync_copy(dsts_hbm.at[pl.ds(e0, L2)], dst0)
    g0 = pltpu.async_copy(h_hbm.at[src0], rows0, sem)
    pltpu.sync_copy(srcs_hbm.at[pl.ds(e0 + L2, L2)], src1)
    pltpu.sync_copy(dsts_hbm.at[pl.ds(e0 + L2, L2)], dst1)
    g1 = pltpu.async_copy(h_hbm.at[src1], rows1, sem)
    g0.wait()
    g1.wait()
    return carry

  lax.fori_loop(0, CH2, chunk, 0)
  plsc.subcore_barrier()
  pltpu.sync_copy(acc.at[pl.ds(s * ZR, ZR)],
                  out_hbm.at[pl.ds(c * NPAD + s * ZR, ZR)])


def _layer1_body(acc_ref, xa_ref, wl_ref, wr_ref, bc_ref,
                 pre_ref, stats_ref, stats_acc):
  a = acc_ref[0] + acc_ref[1]
  cnt = a[:, 4:5]
  m8 = a / jnp.maximum(cnt, 1.0)
  pre = (jnp.dot(m8, wl_ref[...], preferred_element_type=jnp.float32)
         + jnp.dot(xa_ref[...], wr_ref[...], preferred_element_type=jnp.float32)
         + bc_ref[...])
  g = pl.program_id(0)

  @pl.when(g == 0)
  def _():
    stats_acc[...] = jnp.zeros_like(stats_acc)

  s1 = jnp.sum(pre, axis=0, keepdims=True)
  s2 = jnp.sum(pre * pre, axis=0, keepdims=True)
  stats_acc[...] += jnp.concatenate([s1, s2], axis=0)
  pre_ref[0] = pre[:, :32]
  pre_ref[1] = pre[:, 32:]
  stats_ref[...] = stats_acc[...]


def _layer23_body(acc_ref, acc1_ref, h_ref, wl_ref, wr_ref, bc_ref,
                  pre_ref, stats_ref, stats_acc):
  agg = jnp.concatenate([acc_ref[0], acc_ref[1]], axis=1)
  cnt = (acc1_ref[0, :, 4] + acc1_ref[1, :, 4])[:, None]
  hcat = jnp.concatenate([h_ref[0], h_ref[1]], axis=1)
  pre = (jnp.dot(agg / jnp.maximum(cnt, 1.0), wl_ref[...],
                 preferred_element_type=jnp.float32)
         + jnp.dot(hcat, wr_ref[...], preferred_element_type=jnp.float32)
         + bc_ref[...])
  g = pl.program_id(0)

  @pl.when(g == 0)
  def _():
    stats_acc[...] = jnp.zeros_like(stats_acc)

  s1 = jnp.sum(pre, axis=0, keepdims=True)
  s2 = jnp.sum(pre * pre, axis=0, keepdims=True)
  stats_acc[...] += jnp.concatenate([s1, s2], axis=0)
  pre_ref[0] = pre[:, :32]
  pre_ref[1] = pre[:, 32:]
  stats_ref[...] = stats_acc[...]


def _bn_mv(stats):
  m = stats[0:1] / N
  v = stats[1:2] / N - m * m
  return m, jnp.sqrt(v + 1e-5)


def _bn_apply(pre, m, sd, gv, bv):
  return jnp.maximum(gv * (pre - m) / sd + bv, 0.0)


def _bnrelu_body(pre_ref, stats_ref, g_ref, b_ref, h_ref):
  m, sd = _bn_mv(stats_ref[...])
  gv, bv = g_ref[...], b_ref[...]
  h_ref[0] = _bn_apply(pre_ref[0], m[:, :32], sd[:, :32], gv[:, :32], bv[:, :32])
  h_ref[1] = _bn_apply(pre_ref[1], m[:, 32:], sd[:, 32:], gv[:, 32:], bv[:, 32:])


def _pool_body(pre_ref, stats_ref, g_ref, b_ref, batch_ref,
               wf1_ref, bf1_ref, wf2_ref, bf2_ref,
               out_ref, pool_acc, cnt_acc):
  m, sd = _bn_mv(stats_ref[...])
  gv, bv = g_ref[...], b_ref[...]
  h0 = _bn_apply(pre_ref[0], m[:, :32], sd[:, :32], gv[:, :32], bv[:, :32])
  h1 = _bn_apply(pre_ref[1], m[:, 32:], sd[:, 32:], gv[:, 32:], bv[:, 32:])
  hcat = jnp.concatenate([h0, h1], axis=1)
  bb = batch_ref[0, 0, :]
  onehot = (bb[:, None] == lax.broadcasted_iota(jnp.int32, (1, G), 1)
            ).astype(jnp.float32)
  part = lax.dot_general(onehot, hcat, (((0,), (0,)), ((), ())),
                         preferred_element_type=jnp.float32,
                         precision=lax.Precision.HIGHEST)
  g = pl.program_id(0)

  @pl.when(g == 0)
  def _():
    pool_acc[...] = jnp.zeros_like(pool_acc)
    cnt_acc[...] = jnp.zeros_like(cnt_acc)

  pool_acc[...] += part
  cnt_acc[0:1, :] += jnp.sum(onehot, axis=0, keepdims=True)

  @pl.when(g == NB - 1)
  def _():
    cnt = cnt_acc[0, :][:, None]
    pooled = pool_acc[...] / jnp.maximum(cnt, 1.0)
    f = jnp.maximum(
        jnp.dot(pooled, wf1_ref[...], preferred_element_type=jnp.float32)
        + bf1_ref[...], 0.0)
    out_ref[...] = (jnp.dot(f, wf2_ref[...],
                            preferred_element_type=jnp.float32)
                    + bf2_ref[...])


_SEQ = pltpu.CompilerParams(dimension_semantics=("arbitrary",))


def _layer1_tc(acc1, xa, wl8, wr8, bc):
  return pl.pallas_call(
      _layer1_body,
      grid=(NB,),
      in_specs=[
          pl.BlockSpec((2, R, 8), lambda g: (0, g, 0)),
          pl.BlockSpec((R, 8), lambda g: (g, 0)),
          pl.BlockSpec((8, 64), lambda g: (0, 0)),
          pl.BlockSpec((8, 64), lambda g: (0, 0)),
          pl.BlockSpec((1, 64), lambda g: (0, 0)),
      ],
      out_specs=[
          pl.BlockSpec((2, R, 32), lambda g: (0, g, 0)),
          pl.BlockSpec((2, 64), lambda g: (0, 0)),
      ],
      out_shape=[
          jax.ShapeDtypeStruct((2, N, 32), jnp.float32),
          jax.ShapeDtypeStruct((2, 64), jnp.float32),
      ],
      scratch_shapes=[pltpu.VMEM((2, 64), jnp.float32)],
      compiler_params=_SEQ,
  )(acc1, xa, wl8, wr8, bc)


def _layer23_tc(acc, acc1, h, wl, wr, bc):
  return pl.pallas_call(
      _layer23_body,
      grid=(NB,),
      in_specs=[
          pl.BlockSpec((2, R, 32), lambda g: (0, g, 0)),
          pl.BlockSpec((2, R, 8), lambda g: (0, g, 0)),
          pl.BlockSpec((2, R, 32), lambda g: (0, g, 0)),
          pl.BlockSpec((64, 64), lambda g: (0, 0)),
          pl.BlockSpec((64, 64), lambda g: (0, 0)),
          pl.BlockSpec((1, 64), lambda g: (0, 0)),
      ],
      out_specs=[
          pl.BlockSpec((2, R, 32), lambda g: (0, g, 0)),
          pl.BlockSpec((2, 64), lambda g: (0, 0)),
      ],
      out_shape=[
          jax.ShapeDtypeStruct((2, N, 32), jnp.float32),
          jax.ShapeDtypeStruct((2, 64), jnp.float32),
      ],
      scratch_shapes=[pltpu.VMEM((2, 64), jnp.float32)],
      compiler_params=_SEQ,
  )(acc, acc1, h, wl, wr, bc)


def _bnrelu_tc(pre, stats, gv, bv):
  return pl.pallas_call(
      _bnrelu_body,
      grid=(NB,),
      in_specs=[
          pl.BlockSpec((2, R, 32), lambda g: (0, g, 0)),
          pl.BlockSpec((2, 64), lambda g: (0, 0)),
          pl.BlockSpec((1, 64), lambda g: (0, 0)),
          pl.BlockSpec((1, 64), lambda g: (0, 0)),
      ],
      out_specs=pl.BlockSpec((2, R, 32), lambda g: (0, g, 0)),
      out_shape=jax.ShapeDtypeStruct((2, N, 32), jnp.float32),
      compiler_params=_SEQ,
  )(pre, stats, gv, bv)


def _pool_tc(pre, stats, gv, bv, batch3d, wf1, bf1, wf2p, bf2p):
  return pl.pallas_call(
      _pool_body,
      grid=(NB,),
      in_specs=[
          pl.BlockSpec((2, R, 32), lambda g: (0, g, 0)),
          pl.BlockSpec((2, 64), lambda g: (0, 0)),
          pl.BlockSpec((1, 64), lambda g: (0, 0)),
          pl.BlockSpec((1, 64), lambda g: (0, 0)),
          pl.BlockSpec((1, 1, R), lambda g: (g, 0, 0)),
          pl.BlockSpec((64, 64), lambda g: (0, 0)),
          pl.BlockSpec((1, 64), lambda g: (0, 0)),
          pl.BlockSpec((64, 128), lambda g: (0, 0)),
          pl.BlockSpec((1, 128), lambda g: (0, 0)),
      ],
      out_specs=pl.BlockSpec((G, 128), lambda g: (0, 0)),
      out_shape=jax.ShapeDtypeStruct((G, 128), jnp.float32),
      scratch_shapes=[pltpu.VMEM((G, 64), jnp.float32),
                      pltpu.VMEM((8, G), jnp.float32)],
      compiler_params=_SEQ,
  )(pre, stats, gv, bv, batch3d, wf1, bf1, wf2p, bf2p)


def kernel(x, edge_index, batch, Wl1, Wr1, bc1, g1, bb1, Wl2, Wr2, bc2, g2,
           bb2, Wl3, Wr3, bc3, g3, bb3, Wf1, bf1, Wf2, bf2):
  src = edge_index[0].astype(jnp.int32)
  dst = edge_index[1].astype(jnp.int32)
  ar = jnp.arange(PADE, dtype=jnp.int32)
  srcs1d = jnp.concatenate([src, (ar * 37) % N])
  dsts1d = jnp.concatenate([dst, N + (ar % 48)])

  xa = jnp.concatenate(
      [x, jnp.ones((N, 1), jnp.float32), jnp.zeros((N, 3), jnp.float32)],
      axis=1)
  zeros8 = jnp.zeros((NPAD, 8), jnp.float32)
  zeros32 = jnp.zeros((NPAD, 32), jnp.float32)

  wl8 = jnp.concatenate([Wl1, jnp.zeros((4, 64), jnp.float32)], axis=0)
  wr8 = jnp.concatenate([Wr1, jnp.zeros((4, 64), jnp.float32)], axis=0)

  bc1r, bc2r, bc3r = bc1[None], bc2[None], bc3[None]
  g1r, g2r, g3r = g1[None], g2[None], g3[None]
  bb1r, bb2r, bb3r = bb1[None], bb2[None], bb3[None]
  bf1r = bf1[None]
  wf2p = jnp.pad(Wf2, ((0, 0), (0, 126)))
  bf2p = jnp.pad(bf2, (0, 126))[None]
  batch3d = batch.astype(jnp.int32).reshape(NB, 1, R)

  acc1 = _sc_agg8(xa, srcs1d, dsts1d, zeros8).reshape(2, NPAD, 8)
  pre1, stats1 = _layer1_tc(acc1, xa, wl8, wr8, bc1r)
  h1 = _bnrelu_tc(pre1, stats1, g1r, bb1r)

  acc2 = _sc_agg32(h1.reshape(N, 64), srcs1d, dsts1d,
                   zeros32).reshape(2, NPAD, 32)
  pre2, stats2 = _layer23_tc(acc2, acc1, h1, Wl2, Wr2, bc2r)
  h2 = _bnrelu_tc(pre2, stats2, g2r, bb2r)

  acc3 = _sc_agg32(h2.reshape(N, 64), srcs1d, dsts1d,
                   zeros32).reshape(2, NPAD, 32)
  pre3, stats3 = _layer23_tc(acc3, acc1, h2, Wl3, Wr3, bc3r)

  out = _pool_tc(pre3, stats3, g3r, bb3r, batch3d, Wf1, bf1r, wf2p, bf2p)
  return out[:, :2]


# speedup vs baseline: 1.3878x; 1.3878x over previous
import functools

import jax
import jax.numpy as jnp
from jax import lax
from jax.experimental import pallas as pl
from jax.experimental.pallas import tpu as pltpu
from jax.experimental.pallas import tpu_sc as plsc

N = 50000
E = 800000
H = 64
G = 16

NC = 2
NS = 16
NW = NC * NS

EROWS = 6400
E_PAD = EROWS * 128
PADE = E_PAD - E

NPAD = 50048
ZR = NPAD // NS

ROWS1 = EROWS // NW
K1 = 8
CH1 = ROWS1 // K1

ROWS2 = EROWS // NS
K2 = 5
CH2 = ROWS2 // K2

R = 2000
NB = N // R

_MESH = plsc.VectorSubcoreMesh(
    core_axis_name="c", subcore_axis_name="s", num_cores=NC, num_subcores=NS)

_SC_PARAMS = pltpu.CompilerParams(use_tc_tiling_on_sc=False)


@functools.partial(
    pl.kernel,
    out_type=jax.ShapeDtypeStruct((NC * NPAD, 8), jnp.float32),
    mesh=_MESH,
    scratch_types=[
        pltpu.VMEM((K1, 128), jnp.int32),
        pltpu.VMEM((K1, 128), jnp.int32),
        pltpu.VMEM((K1, 128, 8), jnp.float32),
        pltpu.VMEM_SHARED((NPAD, 8), jnp.float32),
        pltpu.SemaphoreType.DMA,
        pltpu.SemaphoreType.DMA,
    ],
    compiler_params=_SC_PARAMS,
)
def _sc_agg8(xa_hbm, srcs_hbm, dsts_hbm, zeros_hbm, out_hbm,
             src_v, dst_v, rows_v, acc, sem, sem2):
  c = lax.axis_index("c")
  s = lax.axis_index("s")
  wid = c * NS + s
  pltpu.sync_copy(zeros_hbm.at[pl.ds(s * ZR, ZR)], acc.at[pl.ds(s * ZR, ZR)])
  plsc.subcore_barrier()
  base = wid * ROWS1

  def chunk(i, carry):
    rb = base + i * K1
    pltpu.sync_copy(srcs_hbm.at[pl.ds(rb, K1)], src_v)
    pltpu.sync_copy(dsts_hbm.at[pl.ds(rb, K1)], dst_v)
    descs = [
        pltpu.async_copy(xa_hbm.at[src_v.at[j]], rows_v.at[j], sem)
        for j in range(K1)
    ]
    sdescs = []
    for j in range(K1):
      descs[j].wait()
      sdescs.append(
          pltpu.async_copy(rows_v.at[j], acc.at[dst_v.at[j]], sem2, add=True))
    for d in sdescs:
      d.wait()
    return carry

  lax.fori_loop(0, CH1, chunk, 0)
  plsc.subcore_barrier()
  pltpu.sync_copy(acc.at[pl.ds(s * ZR, ZR)],
                  out_hbm.at[pl.ds(c * NPAD + s * ZR, ZR)])


@functools.partial(
    pl.kernel,
    out_type=jax.ShapeDtypeStruct((NC * NPAD, 32), jnp.float32),
    mesh=_MESH,
    scratch_types=[
        pltpu.VMEM((K2, 128), jnp.int32),
        pltpu.VMEM((K2, 128), jnp.int32),
        pltpu.VMEM((K2, 128, 32), jnp.float32),
        pltpu.VMEM_SHARED((NPAD, 32), jnp.float32),
        pltpu.SemaphoreType.DMA,
        pltpu.SemaphoreType.DMA,
    ],
    compiler_params=_SC_PARAMS,
)
def _sc_agg32(h_hbm, srcs_hbm, dsts_hbm, zeros_hbm, out_hbm,
              src_v, dst_v, rows_v, acc, sem, sem2):
  c = lax.axis_index("c")
  s = lax.axis_index("s")
  off = c * N
  pltpu.sync_copy(zeros_hbm.at[pl.ds(s * ZR, ZR)], acc.at[pl.ds(s * ZR, ZR)])
  plsc.subcore_barrier()
  base = s * ROWS2

  def chunk(i, carry):
    rb = base + i * K2
    pltpu.sync_copy(srcs_hbm.at[pl.ds(rb, K2)], src_v)
    pltpu.sync_copy(dsts_hbm.at[pl.ds(rb, K2)], dst_v)
    for t in range(K2):
      for k in range(8):
        src_v[t, pl.ds(k * 16, 16)] = src_v[t, pl.ds(k * 16, 16)] + off
    descs = [
        pltpu.async_copy(h_hbm.at[src_v.at[j]], rows_v.at[j], sem)
        for j in range(K2)
    ]
    sdescs = []
    for j in range(K2):
      descs[j].wait()
      sdescs.append(
          pltpu.async_copy(rows_v.at[j], acc.at[dst_v.at[j]], sem2, add=True))
    for d in sdescs:
      d.wait()
    return carry

  lax.fori_loop(0, CH2, chunk, 0)
  plsc.subcore_barrier()
  pltpu.sync_copy(acc.at[pl.ds(s * ZR, ZR)],
                  out_hbm.at[pl.ds(c * NPAD + s * ZR, ZR)])


def _bn_mv(stats):
  m = stats[0:1] / N
  v = stats[1:2] / N - m * m
  return m, jnp.sqrt(v + 1e-5)


def _bn_apply(pre, m, sd, gv, bv):
  return jnp.maximum(gv * (pre - m) / sd + bv, 0.0)


def _accum_stats(pre, p, g, stats_acc):
  @pl.when(jnp.logical_and(p == 0, g == 0))
  def _():
    stats_acc[...] = jnp.zeros_like(stats_acc)

  @pl.when(p == 0)
  def _():
    s1 = jnp.sum(pre, axis=0, keepdims=True)
    s2 = jnp.sum(pre * pre, axis=0, keepdims=True)
    stats_acc[...] += jnp.concatenate([s1, s2], axis=0)


def _layer1_body(acc_ref, xa_ref, wl_ref, wr_ref, bc_ref, g_ref, b_ref,
                 h_ref, pre_buf, stats_acc):
  p = pl.program_id(0)
  g = pl.program_id(1)

  @pl.when(p == 0)
  def _():
    a = acc_ref[0] + acc_ref[1]
    cnt = a[:, 4:5]
    m8 = a / jnp.maximum(cnt, 1.0)
    pre = (jnp.dot(m8, wl_ref[...], preferred_element_type=jnp.float32)
           + jnp.dot(xa_ref[...], wr_ref[...],
                     preferred_element_type=jnp.float32)
           + bc_ref[...])
    pre_buf[pl.ds(g * R, R), :] = pre
  _accum_stats(pre_buf[pl.ds(g * R, R), :], p, g, stats_acc)

  @pl.when(p == 1)
  def _():
    m, sd = _bn_mv(stats_acc[...])
    pre = pre_buf[pl.ds(g * R, R), :]
    h = _bn_apply(pre, m, sd, g_ref[...], b_ref[...])
    h_ref[0] = h[:, :32]
    h_ref[1] = h[:, 32:]


def _layer2_body(acc_ref, acc1_ref, h_ref, wl_ref, wr_ref, bc_ref,
                 g_ref, b_ref, hout_ref, pre_buf, stats_acc):
  p = pl.program_id(0)
  g = pl.program_id(1)

  @pl.when(p == 0)
  def _():
    agg = jnp.concatenate([acc_ref[0], acc_ref[1]], axis=1)
    cnt = (acc1_ref[0, :, 4] + acc1_ref[1, :, 4])[:, None]
    hcat = jnp.concatenate([h_ref[0], h_ref[1]], axis=1)
    pre = (jnp.dot(agg / jnp.maximum(cnt, 1.0), wl_ref[...],
                   preferred_element_type=jnp.float32)
           + jnp.dot(hcat, wr_ref[...], preferred_element_type=jnp.float32)
           + bc_ref[...])
    pre_buf[pl.ds(g * R, R), :] = pre
  _accum_stats(pre_buf[pl.ds(g * R, R), :], p, g, stats_acc)

  @pl.when(p == 1)
  def _():
    m, sd = _bn_mv(stats_acc[...])
    pre = pre_buf[pl.ds(g * R, R), :]
    h = _bn_apply(pre, m, sd, g_ref[...], b_ref[...])
    hout_ref[0] = h[:, :32]
    hout_ref[1] = h[:, 32:]


def _layer3_body(acc_ref, acc1_ref, h_ref, wl_ref, wr_ref, bc_ref,
                 g_ref, b_ref, batch_ref, wf1_ref, bf1_ref, wf2_ref, bf2_ref,
                 out_ref, pre_buf, stats_acc, pool_acc, cnt_acc):
  p = pl.program_id(0)
  g = pl.program_id(1)

  @pl.when(p == 0)
  def _():
    agg = jnp.concatenate([acc_ref[0], acc_ref[1]], axis=1)
    cnt = (acc1_ref[0, :, 4] + acc1_ref[1, :, 4])[:, None]
    hcat = jnp.concatenate([h_ref[0], h_ref[1]], axis=1)
    pre = (jnp.dot(agg / jnp.maximum(cnt, 1.0), wl_ref[...],
                   preferred_element_type=jnp.float32)
           + jnp.dot(hcat, wr_ref[...], preferred_element_type=jnp.float32)
           + bc_ref[...])
    pre_buf[pl.ds(g * R, R), :] = pre
  _accum_stats(pre_buf[pl.ds(g * R, R), :], p, g, stats_acc)

  @pl.when(p == 1)
  def _():
    m, sd = _bn_mv(stats_acc[...])
    pre = pre_buf[pl.ds(g * R, R), :]
    h3 = _bn_apply(pre, m, sd, g_ref[...], b_ref[...])
    bb = batch_ref[0, 0, :]
    onehot = (bb[:, None] == lax.broadcasted_iota(jnp.int32, (1, G), 1)
              ).astype(jnp.float32)
    part = lax.dot_general(onehot, h3, (((0,), (0,)), ((), ())),
                           preferred_element_type=jnp.float32,
                           precision=lax.Precision.HIGHEST)

    @pl.when(g == 0)
    def _():
      pool_acc[...] = jnp.zeros_like(pool_acc)
      cnt_acc[...] = jnp.zeros_like(cnt_acc)

    pool_acc[...] += part
    cnt_acc[0:1, :] += jnp.sum(onehot, axis=0, keepdims=True)

    @pl.when(g == NB - 1)
    def _():
      cntg = cnt_acc[0, :][:, None]
      pooled = pool_acc[...] / jnp.maximum(cntg, 1.0)
      f = jnp.maximum(
          jnp.dot(pooled, wf1_ref[...], preferred_element_type=jnp.float32)
          + bf1_ref[...], 0.0)
      out_ref[...] = (jnp.dot(f, wf2_ref[...],
                              preferred_element_type=jnp.float32)
                      + bf2_ref[...])


_SEQ = pltpu.CompilerParams(dimension_semantics=("arbitrary", "arbitrary"))


def _p0(bs):
  return pl.BlockSpec(bs, lambda p, g: (0, g * (1 - p), 0))


def _cst(bs):
  nd = len(bs)
  return pl.BlockSpec(bs, lambda p, g, _n=nd: (0,) * _n)


def _layer1_tc(acc1, xa, wl8, wr8, bc, gv, bv):
  return pl.pallas_call(
      _layer1_body,
      grid=(2, NB),
      in_specs=[
          _p0((2, R, 8)),
          pl.BlockSpec((R, 8), lambda p, g: (g * (1 - p), 0)),
          _cst((8, 64)),
          _cst((8, 64)),
          _cst((1, 64)),
          _cst((1, 64)),
          _cst((1, 64)),
      ],
      out_specs=pl.BlockSpec((2, R, 32), lambda p, g: (0, g * p, 0)),
      out_shape=jax.ShapeDtypeStruct((2, N, 32), jnp.float32),
      scratch_shapes=[pltpu.VMEM((N, 64), jnp.float32),
                      pltpu.VMEM((2, 64), jnp.float32)],
      compiler_params=_SEQ,
  )(acc1, xa, wl8, wr8, bc, gv, bv)


def _layer2_tc(acc, acc1, h, wl, wr, bc, gv, bv):
  return pl.pallas_call(
      _layer2_body,
      grid=(2, NB),
      in_specs=[
          _p0((2, R, 32)),
          _p0((2, R, 8)),
          _p0((2, R, 32)),
          _cst((64, 64)),
          _cst((64, 64)),
          _cst((1, 64)),
          _cst((1, 64)),
          _cst((1, 64)),
      ],
      out_specs=pl.BlockSpec((2, R, 32), lambda p, g: (0, g * p, 0)),
      out_shape=jax.ShapeDtypeStruct((2, N, 32), jnp.float32),
      scratch_shapes=[pltpu.VMEM((N, 64), jnp.float32),
                      pltpu.VMEM((2, 64), jnp.float32)],
      compiler_params=_SEQ,
  )(acc, acc1, h, wl, wr, bc, gv, bv)


def _layer3_tc(acc, acc1, h, wl, wr, bc, gv, bv, batch3d, wf1, bf1, wf2p,
               bf2p):
  return pl.pallas_call(
      _layer3_body,
      grid=(2, NB),
      in_specs=[
          _p0((2, R, 32)),
          _p0((2, R, 8)),
          _p0((2, R, 32)),
          _cst((64, 64)),
          _cst((64, 64)),
          _cst((1, 64)),
          _cst((1, 64)),
          _cst((1, 64)),
          pl.BlockSpec((1, 1, R), lambda p, g: (g * p, 0, 0)),
          _cst((64, 64)),
          _cst((1, 64)),
          _cst((64, 128)),
          _cst((1, 128)),
      ],
      out_specs=pl.BlockSpec((G, 128), lambda p, g: (0, 0)),
      out_shape=jax.ShapeDtypeStruct((G, 128), jnp.float32),
      scratch_shapes=[pltpu.VMEM((N, 64), jnp.float32),
                      pltpu.VMEM((2, 64), jnp.float32),
                      pltpu.VMEM((G, 64), jnp.float32),
                      pltpu.VMEM((8, G), jnp.float32)],
      compiler_params=_SEQ,
  )(acc, acc1, h, wl, wr, bc, gv, bv, batch3d, wf1, bf1, wf2p, bf2p)


def kernel(x, edge_index, batch, Wl1, Wr1, bc1, g1, bb1, Wl2, Wr2, bc2, g2,
           bb2, Wl3, Wr3, bc3, g3, bb3, Wf1, bf1, Wf2, bf2):
  src = edge_index[0].astype(jnp.int32)
  dst = edge_index[1].astype(jnp.int32)
  ar = jnp.arange(PADE, dtype=jnp.int32)
  srcs2d = jnp.concatenate([src, (ar * 37) % N]).reshape(EROWS, 128)
  dsts2d = jnp.concatenate([dst, N + (ar % 48)]).reshape(EROWS, 128)

  xa = jnp.concatenate(
      [x, jnp.ones((N, 1), jnp.float32), jnp.zeros((N, 3), jnp.float32)],
      axis=1)
  zeros8 = jnp.zeros((NPAD, 8), jnp.float32)
  zeros32 = jnp.zeros((NPAD, 32), jnp.float32)

  wl8 = jnp.concatenate([Wl1, jnp.zeros((4, 64), jnp.float32)], axis=0)
  wr8 = jnp.concatenate([Wr1, jnp.zeros((4, 64), jnp.float32)], axis=0)

  bc1r, bc2r, bc3r = bc1[None], bc2[None], bc3[None]
  g1r, g2r, g3r = g1[None], g2[None], g3[None]
  bb1r, bb2r, bb3r = bb1[None], bb2[None], bb3[None]
  bf1r = bf1[None]
  wf2p = jnp.pad(Wf2, ((0, 0), (0, 126)))
  bf2p = jnp.pad(bf2, (0, 126))[None]
  batch3d = batch.astype(jnp.int32).reshape(NB, 1, R)

  acc1 = _sc_agg8(xa, srcs2d, dsts2d, zeros8).reshape(2, NPAD, 8)
  h1 = _layer1_tc(acc1, xa, wl8, wr8, bc1r, g1r, bb1r)

  acc2 = _sc_agg32(h1.reshape(2 * N, 32), srcs2d, dsts2d,
                   zeros32).reshape(2, NPAD, 32)
  h2 = _layer2_tc(acc2, acc1, h1, Wl2, Wr2, bc2r, g2r, bb2r)

  acc3 = _sc_agg32(h2.reshape(2 * N, 32), srcs2d, dsts2d,
                   zeros32).reshape(2, NPAD, 32)
  out = _layer3_tc(acc3, acc1, h2, Wl3, Wr3, bc3r, g3r, bb3r, batch3d,
                   Wf1, bf1r, wf2p, bf2p)
  return out[:, :2]


# speedup vs baseline: 1.3991x; 1.0082x over previous
import functools

import jax
import jax.numpy as jnp
from jax import lax
from jax.experimental import pallas as pl
from jax.experimental.pallas import tpu as pltpu
from jax.experimental.pallas import tpu_sc as plsc

N = 50000
E = 800000
H = 64
G = 16

NC = 2
NS = 16
NW = NC * NS

EROWS = 6400
E_PAD = EROWS * 128
PADE = E_PAD - E

NPAD = 50048
ZR = NPAD // NS

ROWS1 = EROWS // NW
K1 = 10
CH1 = ROWS1 // K1

ROWS2 = EROWS // NS
K2 = 5
CH2 = ROWS2 // K2

R = 2000
NB = N // R

_MESH = plsc.VectorSubcoreMesh(
    core_axis_name="c", subcore_axis_name="s", num_cores=NC, num_subcores=NS)

_SC_PARAMS = pltpu.CompilerParams(use_tc_tiling_on_sc=False)


@functools.partial(
    pl.kernel,
    out_type=jax.ShapeDtypeStruct((NC * NPAD, 8), jnp.float32),
    mesh=_MESH,
    scratch_types=[
        pltpu.VMEM((K1, 128), jnp.int32),
        pltpu.VMEM((K1, 128), jnp.int32),
        pltpu.VMEM((K1, 128, 8), jnp.float32),
        pltpu.VMEM_SHARED((NPAD, 8), jnp.float32),
        pltpu.SemaphoreType.DMA,
        pltpu.SemaphoreType.DMA,
    ],
    compiler_params=_SC_PARAMS,
)
def _sc_agg8(xa_hbm, srcs_hbm, dsts_hbm, zeros_hbm, out_hbm,
             src_v, dst_v, rows_v, acc, sem, sem2):
  c = lax.axis_index("c")
  s = lax.axis_index("s")
  wid = c * NS + s
  pltpu.sync_copy(zeros_hbm.at[pl.ds(s * ZR, ZR)], acc.at[pl.ds(s * ZR, ZR)])
  plsc.subcore_barrier()
  base = wid * ROWS1

  def chunk(i, carry):
    rb = base + i * K1
    pltpu.sync_copy(srcs_hbm.at[pl.ds(rb, K1)], src_v)
    pltpu.sync_copy(dsts_hbm.at[pl.ds(rb, K1)], dst_v)
    descs = [
        pltpu.async_copy(xa_hbm.at[src_v.at[j]], rows_v.at[j], sem)
        for j in range(K1)
    ]
    sdescs = []
    for j in range(K1):
      descs[j].wait()
      sdescs.append(
          pltpu.async_copy(rows_v.at[j], acc.at[dst_v.at[j]], sem2, add=True))
    for d in sdescs:
      d.wait()
    return carry

  lax.fori_loop(0, CH1, chunk, 0)
  plsc.subcore_barrier()
  pltpu.sync_copy(acc.at[pl.ds(s * ZR, ZR)],
                  out_hbm.at[pl.ds(c * NPAD + s * ZR, ZR)])


@functools.partial(
    pl.kernel,
    out_type=jax.ShapeDtypeStruct((NC * NPAD, 32), jnp.float32),
    mesh=_MESH,
    scratch_types=[
        pltpu.VMEM((K2, 128), jnp.int32),
        pltpu.VMEM((K2, 128), jnp.int32),
        pltpu.VMEM((K2, 128, 32), jnp.float32),
        pltpu.VMEM_SHARED((NPAD, 32), jnp.float32),
        pltpu.SemaphoreType.DMA,
        pltpu.SemaphoreType.DMA,
    ],
    compiler_params=_SC_PARAMS,
)
def _sc_agg32(h_hbm, srcs_hbm, dsts_hbm, zeros_hbm, out_hbm,
              src_v, dst_v, rows_v, acc, sem, sem2):
  c = lax.axis_index("c")
  s = lax.axis_index("s")
  off = c * N
  pltpu.sync_copy(zeros_hbm.at[pl.ds(s * ZR, ZR)], acc.at[pl.ds(s * ZR, ZR)])
  plsc.subcore_barrier()
  base = s * ROWS2

  def chunk(i, carry):
    rb = base + i * K2
    pltpu.sync_copy(srcs_hbm.at[pl.ds(rb, K2)], src_v)
    pltpu.sync_copy(dsts_hbm.at[pl.ds(rb, K2)], dst_v)
    for t in range(K2):
      for k in range(8):
        src_v[t, pl.ds(k * 16, 16)] = src_v[t, pl.ds(k * 16, 16)] + off
    descs = [
        pltpu.async_copy(h_hbm.at[src_v.at[j]], rows_v.at[j], sem)
        for j in range(K2)
    ]
    sdescs = []
    for j in range(K2):
      descs[j].wait()
      sdescs.append(
          pltpu.async_copy(rows_v.at[j], acc.at[dst_v.at[j]], sem2, add=True))
    for d in sdescs:
      d.wait()
    return carry

  lax.fori_loop(0, CH2, chunk, 0)
  plsc.subcore_barrier()
  pltpu.sync_copy(acc.at[pl.ds(s * ZR, ZR)],
                  out_hbm.at[pl.ds(c * NPAD + s * ZR, ZR)])


def _bn_mv(stats):
  m = stats[0:1] / N
  v = stats[1:2] / N - m * m
  return m, jnp.sqrt(v + 1e-5)


def _bn_apply(pre, m, sd, gv, bv):
  return jnp.maximum(gv * (pre - m) / sd + bv, 0.0)


def _accum_stats(pre, p, g, stats_acc):
  @pl.when(jnp.logical_and(p == 0, g == 0))
  def _():
    stats_acc[...] = jnp.zeros_like(stats_acc)

  @pl.when(p == 0)
  def _():
    s1 = jnp.sum(pre, axis=0, keepdims=True)
    s2 = jnp.sum(pre * pre, axis=0, keepdims=True)
    stats_acc[...] += jnp.concatenate([s1, s2], axis=0)


def _layer1_body(acc_ref, xa_ref, wl_ref, wr_ref, bc_ref, g_ref, b_ref,
                 h_ref, pre_buf, stats_acc):
  p = pl.program_id(0)
  g = pl.program_id(1)

  @pl.when(p == 0)
  def _():
    a = acc_ref[0] + acc_ref[1]
    cnt = a[:, 4:5]
    m8 = a / jnp.maximum(cnt, 1.0)
    pre = (jnp.dot(m8, wl_ref[...], preferred_element_type=jnp.float32)
           + jnp.dot(xa_ref[...], wr_ref[...],
                     preferred_element_type=jnp.float32)
           + bc_ref[...])
    pre_buf[pl.ds(g * R, R), :] = pre
  _accum_stats(pre_buf[pl.ds(g * R, R), :], p, g, stats_acc)

  @pl.when(p == 1)
  def _():
    m, sd = _bn_mv(stats_acc[...])
    pre = pre_buf[pl.ds(g * R, R), :]
    h = _bn_apply(pre, m, sd, g_ref[...], b_ref[...])
    h_ref[0] = h[:, :32]
    h_ref[1] = h[:, 32:]


def _layer2_body(acc_ref, acc1_ref, h_ref, wl_ref, wr_ref, bc_ref,
                 g_ref, b_ref, hout_ref, pre_buf, stats_acc):
  p = pl.program_id(0)
  g = pl.program_id(1)

  @pl.when(p == 0)
  def _():
    agg = jnp.concatenate([acc_ref[0], acc_ref[1]], axis=1)
    cnt = (acc1_ref[0, :, 4] + acc1_ref[1, :, 4])[:, None]
    hcat = jnp.concatenate([h_ref[0], h_ref[1]], axis=1)
    pre = (jnp.dot(agg / jnp.maximum(cnt, 1.0), wl_ref[...],
                   preferred_element_type=jnp.float32)
           + jnp.dot(hcat, wr_ref[...], preferred_element_type=jnp.float32)
           + bc_ref[...])
    pre_buf[pl.ds(g * R, R), :] = pre
  _accum_stats(pre_buf[pl.ds(g * R, R), :], p, g, stats_acc)

  @pl.when(p == 1)
  def _():
    m, sd = _bn_mv(stats_acc[...])
    pre = pre_buf[pl.ds(g * R, R), :]
    h = _bn_apply(pre, m, sd, g_ref[...], b_ref[...])
    hout_ref[0] = h[:, :32]
    hout_ref[1] = h[:, 32:]


def _layer3_body(acc_ref, acc1_ref, h_ref, wl_ref, wr_ref, bc_ref,
                 g_ref, b_ref, batch_ref, wf1_ref, bf1_ref, wf2_ref, bf2_ref,
                 out_ref, pre_buf, stats_acc, pool_acc, cnt_acc):
  p = pl.program_id(0)
  g = pl.program_id(1)

  @pl.when(p == 0)
  def _():
    agg = jnp.concatenate([acc_ref[0], acc_ref[1]], axis=1)
    cnt = (acc1_ref[0, :, 4] + acc1_ref[1, :, 4])[:, None]
    hcat = jnp.concatenate([h_ref[0], h_ref[1]], axis=1)
    pre = (jnp.dot(agg / jnp.maximum(cnt, 1.0), wl_ref[...],
                   preferred_element_type=jnp.float32)
           + jnp.dot(hcat, wr_ref[...], preferred_element_type=jnp.float32)
           + bc_ref[...])
    pre_buf[pl.ds(g * R, R), :] = pre
  _accum_stats(pre_buf[pl.ds(g * R, R), :], p, g, stats_acc)

  @pl.when(p == 1)
  def _():
    m, sd = _bn_mv(stats_acc[...])
    pre = pre_buf[pl.ds(g * R, R), :]
    h3 = _bn_apply(pre, m, sd, g_ref[...], b_ref[...])
    bb = batch_ref[0, 0, :]
    onehot = (bb[:, None] == lax.broadcasted_iota(jnp.int32, (1, G), 1)
              ).astype(jnp.float32)
    part = lax.dot_general(onehot, h3, (((0,), (0,)), ((), ())),
                           preferred_element_type=jnp.float32,
                           precision=lax.Precision.HIGHEST)

    @pl.when(g == 0)
    def _():
      pool_acc[...] = jnp.zeros_like(pool_acc)
      cnt_acc[...] = jnp.zeros_like(cnt_acc)

    pool_acc[...] += part
    cnt_acc[0:1, :] += jnp.sum(onehot, axis=0, keepdims=True)

    @pl.when(g == NB - 1)
    def _():
      cntg = cnt_acc[0, :][:, None]
      pooled = pool_acc[...] / jnp.maximum(cntg, 1.0)
      f = jnp.maximum(
          jnp.dot(pooled, wf1_ref[...], preferred_element_type=jnp.float32)
          + bf1_ref[...], 0.0)
      out_ref[...] = (jnp.dot(f, wf2_ref[...],
                              preferred_element_type=jnp.float32)
                      + bf2_ref[...])


_SEQ = pltpu.CompilerParams(dimension_semantics=("arbitrary", "arbitrary"))


def _p0(bs):
  return pl.BlockSpec(bs, lambda p, g: (0, g * (1 - p), 0))


def _cst(bs):
  nd = len(bs)
  return pl.BlockSpec(bs, lambda p, g, _n=nd: (0,) * _n)


def _layer1_tc(acc1, xa, wl8, wr8, bc, gv, bv):
  return pl.pallas_call(
      _layer1_body,
      grid=(2, NB),
      in_specs=[
          _p0((2, R, 8)),
          pl.BlockSpec((R, 8), lambda p, g: (g * (1 - p), 0)),
          _cst((8, 64)),
          _cst((8, 64)),
          _cst((1, 64)),
          _cst((1, 64)),
          _cst((1, 64)),
      ],
      out_specs=pl.BlockSpec((2, R, 32), lambda p, g: (0, g * p, 0)),
      out_shape=jax.ShapeDtypeStruct((2, N, 32), jnp.float32),
      scratch_shapes=[pltpu.VMEM((N, 64), jnp.float32),
                      pltpu.VMEM((2, 64), jnp.float32)],
      compiler_params=_SEQ,
  )(acc1, xa, wl8, wr8, bc, gv, bv)


def _layer2_tc(acc, acc1, h, wl, wr, bc, gv, bv):
  return pl.pallas_call(
      _layer2_body,
      grid=(2, NB),
      in_specs=[
          _p0((2, R, 32)),
          _p0((2, R, 8)),
          _p0((2, R, 32)),
          _cst((64, 64)),
          _cst((64, 64)),
          _cst((1, 64)),
          _cst((1, 64)),
          _cst((1, 64)),
      ],
      out_specs=pl.BlockSpec((2, R, 32), lambda p, g: (0, g * p, 0)),
      out_shape=jax.ShapeDtypeStruct((2, N, 32), jnp.float32),
      scratch_shapes=[pltpu.VMEM((N, 64), jnp.float32),
                      pltpu.VMEM((2, 64), jnp.float32)],
      compiler_params=_SEQ,
  )(acc, acc1, h, wl, wr, bc, gv, bv)


def _layer3_tc(acc, acc1, h, wl, wr, bc, gv, bv, batch3d, wf1, bf1, wf2p,
               bf2p):
  return pl.pallas_call(
      _layer3_body,
      grid=(2, NB),
      in_specs=[
          _p0((2, R, 32)),
          _p0((2, R, 8)),
          _p0((2, R, 32)),
          _cst((64, 64)),
          _cst((64, 64)),
          _cst((1, 64)),
          _cst((1, 64)),
          _cst((1, 64)),
          pl.BlockSpec((1, 1, R), lambda p, g: (g * p, 0, 0)),
          _cst((64, 64)),
          _cst((1, 64)),
          _cst((64, 128)),
          _cst((1, 128)),
      ],
      out_specs=pl.BlockSpec((G, 128), lambda p, g: (0, 0)),
      out_shape=jax.ShapeDtypeStruct((G, 128), jnp.float32),
      scratch_shapes=[pltpu.VMEM((N, 64), jnp.float32),
                      pltpu.VMEM((2, 64), jnp.float32),
                      pltpu.VMEM((G, 64), jnp.float32),
                      pltpu.VMEM((8, G), jnp.float32)],
      compiler_params=_SEQ,
  )(acc, acc1, h, wl, wr, bc, gv, bv, batch3d, wf1, bf1, wf2p, bf2p)


def kernel(x, edge_index, batch, Wl1, Wr1, bc1, g1, bb1, Wl2, Wr2, bc2, g2,
           bb2, Wl3, Wr3, bc3, g3, bb3, Wf1, bf1, Wf2, bf2):
  src = edge_index[0].astype(jnp.int32)
  dst = edge_index[1].astype(jnp.int32)
  ar = jnp.arange(PADE, dtype=jnp.int32)
  srcs2d = jnp.concatenate([src, (ar * 37) % N]).reshape(EROWS, 128)
  dsts2d = jnp.concatenate([dst, N + (ar % 48)]).reshape(EROWS, 128)

  xa = jnp.concatenate(
      [x, jnp.ones((N, 1), jnp.float32), jnp.zeros((N, 3), jnp.float32)],
      axis=1)
  zeros8 = jnp.zeros((NPAD, 8), jnp.float32)
  zeros32 = jnp.zeros((NPAD, 32), jnp.float32)

  wl8 = jnp.concatenate([Wl1, jnp.zeros((4, 64), jnp.float32)], axis=0)
  wr8 = jnp.concatenate([Wr1, jnp.zeros((4, 64), jnp.float32)], axis=0)

  bc1r, bc2r, bc3r = bc1[None], bc2[None], bc3[None]
  g1r, g2r, g3r = g1[None], g2[None], g3[None]
  bb1r, bb2r, bb3r = bb1[None], bb2[None], bb3[None]
  bf1r = bf1[None]
  wf2p = jnp.pad(Wf2, ((0, 0), (0, 126)))
  bf2p = jnp.pad(bf2, (0, 126))[None]
  batch3d = batch.astype(jnp.int32).reshape(NB, 1, R)

  acc1 = _sc_agg8(xa, srcs2d, dsts2d, zeros8).reshape(2, NPAD, 8)
  h1 = _layer1_tc(acc1, xa, wl8, wr8, bc1r, g1r, bb1r)

  acc2 = _sc_agg32(h1.reshape(2 * N, 32), srcs2d, dsts2d,
                   zeros32).reshape(2, NPAD, 32)
  h2 = _layer2_tc(acc2, acc1, h1, Wl2, Wr2, bc2r, g2r, bb2r)

  acc3 = _sc_agg32(h2.reshape(2 * N, 32), srcs2d, dsts2d,
                   zeros32).reshape(2, NPAD, 32)
  out = _layer3_tc(acc3, acc1, h2, Wl3, Wr3, bc3r, g3r, bb3r, batch3d,
                   Wf1, bf1r, wf2p, bf2p)
  return out[:, :2]
